# baseline (speedup 1.0000x reference)
import functools

import jax
import jax.numpy as jnp
from jax.experimental import pallas as pl
from jax.experimental.pallas import tpu as pltpu

_E = 8
_K = 2
_H = 1024
_I = 2816
_T = 2048
_J = 11
_IJ = _I // _J
_CB = 256


def _router_weights(logits, e):
    probs = jax.nn.softmax(logits, axis=-1)
    ids = jax.lax.broadcasted_iota(jnp.int32, probs.shape, 1)
    top1 = jnp.max(probs, axis=-1, keepdims=True)
    i1 = jnp.min(jnp.where(probs == top1, ids, _E), axis=-1, keepdims=True)
    masked = jnp.where(ids == i1, -jnp.inf, probs)
    top2 = jnp.max(masked, axis=-1, keepdims=True)
    i2 = jnp.min(jnp.where(masked == top2, ids, _E), axis=-1, keepdims=True)
    denom = top1 + top2
    w1 = top1 / denom
    w2 = top2 / denom
    return jnp.where(i1 == e, w1, 0.0) + jnp.where(i2 == e, w2, 0.0)


def _dense_body(rl_ref, x_ref, g_ref, u_ref, d_ref, o_ref):
    e = pl.program_id(0)
    j = pl.program_id(1)
    first = (e == 0) & (j == 0)
    g = g_ref[0]
    u = u_ref[0]
    d = d_ref[0]

    def chunk(c, _):
        sl = pl.ds(c * _CB, _CB)
        xb = x_ref[sl, :]
        w_e = _router_weights(rl_ref[sl, :], e)
        gg = jnp.dot(xb, g, preferred_element_type=jnp.float32)
        uu = jnp.dot(xb, u, preferred_element_type=jnp.float32)
        h = (gg * jax.lax.logistic(gg)) * uu
        y = jnp.dot(h, d, preferred_element_type=jnp.float32)
        contrib = w_e * y
        o_ref[sl, :] = jnp.where(first, contrib, o_ref[sl, :] + contrib)
        return 0

    jax.lax.fori_loop(0, _T // _CB, chunk, 0)


def kernel(x, router_logits, gate_proj, up_proj, down_proj):
    return pl.pallas_call(
        _dense_body,
        grid=(_E, _J),
        in_specs=[
            pl.BlockSpec((_T, _E), lambda e, j: (0, 0)),
            pl.BlockSpec((_T, _H), lambda e, j: (0, 0)),
            pl.BlockSpec((1, _H, _IJ), lambda e, j: (e, 0, j)),
            pl.BlockSpec((1, _H, _IJ), lambda e, j: (e, 0, j)),
            pl.BlockSpec((1, _IJ, _H), lambda e, j: (e, j, 0)),
        ],
        out_specs=pl.BlockSpec((_T, _H), lambda e, j: (0, 0)),
        out_shape=jax.ShapeDtypeStruct((_T, _H), jnp.float32),
        compiler_params=pltpu.CompilerParams(
            dimension_semantics=("arbitrary", "arbitrary"),
        ),
    )(router_logits, x, gate_proj, up_proj, down_proj)
